# initial kernel scaffold (unmeasured)
import jax
import jax.numpy as jnp
from jax import lax
from jax.experimental import pallas as pl
from jax.experimental.pallas import tpu as pltpu

T = 32


def kernel(x, A, B, C):
    b, s_loc, d = x.shape
    n = B.shape[-1]
    n_chunks = s_loc // T

    def body(x_ref, A_ref, B_ref, C_ref, out_ref, h0_ref, hf_ref, send_sem, recv_sem):
        my_x = lax.axis_index("x")
        my_y = lax.axis_index("y")
        nbr = (1 - my_x, my_y)

        barrier = pltpu.get_barrier_semaphore()
        pl.semaphore_signal(
            barrier, inc=1, device_id=nbr, device_id_type=pl.DeviceIdType.MESH
        )
        pl.semaphore_wait(barrier, 1)

        At = A_ref[:, :].astype(jnp.float32).T
        tau = lax.broadcasted_iota(jnp.float32, (T, 1, 1), 0)
        Ep = jnp.exp(At[None] * tau)
        Em = jnp.exp(At[None] * (-tau))
        Ep1 = Ep * jnp.exp(At)[None]

        @pl.when(my_x == 0)
        def _():
            h0_ref[...] = jnp.zeros((b, n, d), jnp.float32)

        @pl.when(my_x == 1)
        def _():
            recv = pltpu.make_async_remote_copy(
                src_ref=hf_ref,
                dst_ref=h0_ref,
                send_sem=send_sem,
                recv_sem=recv_sem,
                device_id=nbr,
                device_id_type=pl.DeviceIdType.MESH,
            )
            recv.wait_recv()

        def chunk(k, h_prev):
            sl = pl.ds(k * T, T)
            x_c = x_ref[:, sl, :].astype(jnp.float32)
            B_c = B_ref[:, sl, :].astype(jnp.float32)
            C_c = C_ref[:, sl, :].astype(jnp.float32)
            U = x_c[:, :, None, :] * B_c[:, :, :, None] * Em[None]
            S = jnp.cumsum(U, axis=1)
            h = Ep[None] * S + Ep1[None] * h_prev[:, None]
            y_c = jnp.sum(h * C_c[:, :, :, None], axis=2)
            out_ref[:, sl, :] = y_c
            return h[:, T - 1]

        h_fin = lax.fori_loop(0, n_chunks, chunk, h0_ref[...])
        hf_ref[...] = h_fin

        @pl.when(my_x == 0)
        def _():
            send = pltpu.make_async_remote_copy(
                src_ref=hf_ref,
                dst_ref=h0_ref,
                send_sem=send_sem,
                recv_sem=recv_sem,
                device_id=nbr,
                device_id_type=pl.DeviceIdType.MESH,
            )
            send.start()
            send.wait_send()

    return pl.pallas_call(
        body,
        out_shape=jax.ShapeDtypeStruct((b, s_loc, d), jnp.float32),
        in_specs=[pl.BlockSpec(memory_space=pltpu.VMEM)] * 4,
        out_specs=pl.BlockSpec(memory_space=pltpu.VMEM),
        scratch_shapes=[
            pltpu.VMEM((b, n, d), jnp.float32),
            pltpu.VMEM((b, n, d), jnp.float32),
            pltpu.SemaphoreType.DMA,
            pltpu.SemaphoreType.DMA,
        ],
        compiler_params=pltpu.CompilerParams(
            collective_id=0, vmem_limit_bytes=110 * 1024 * 1024
        ),
    )(x, A, B, C)


# baseline (device time: 168941 ns/iter reference)
import jax
import jax.numpy as jnp
from jax import lax
from jax.experimental import pallas as pl
from jax.experimental.pallas import tpu as pltpu

T = 32


def kernel(x, A, B, C):
    b, s_loc, d = x.shape
    n = B.shape[-1]
    n_chunks = s_loc // T

    def body(x_ref, A_ref, B_ref, C_ref, out_ref, h0_ref, hf_ref, send_sem, recv_sem):
        my_x = lax.axis_index("x")
        my_y = lax.axis_index("y")
        nbr = (1 - my_x, my_y)

        barrier = pltpu.get_barrier_semaphore()
        pl.semaphore_signal(
            barrier, inc=1, device_id=nbr, device_id_type=pl.DeviceIdType.MESH
        )
        pl.semaphore_wait(barrier, 1)

        At = A_ref[:, :].astype(jnp.float32).T
        tau = lax.broadcasted_iota(jnp.int32, (T, 1, 1), 0).astype(jnp.float32)
        Ep = jnp.exp(At[None] * tau)
        Em = jnp.exp(At[None] * (-tau))
        Ep1 = Ep * jnp.exp(At)[None]

        @pl.when(my_x == 0)
        def _():
            h0_ref[...] = jnp.zeros((b, n, d), jnp.float32)

        @pl.when(my_x == 1)
        def _():
            recv = pltpu.make_async_remote_copy(
                src_ref=hf_ref,
                dst_ref=h0_ref,
                send_sem=send_sem,
                recv_sem=recv_sem,
                device_id=nbr,
                device_id_type=pl.DeviceIdType.MESH,
            )
            recv.wait_recv()

        def chunk(k, h_prev):
            sl = pl.ds(k * T, T)
            x_c = x_ref[:, sl, :].astype(jnp.float32)
            B_c = B_ref[:, sl, :].astype(jnp.float32)
            C_c = C_ref[:, sl, :].astype(jnp.float32)
            U = x_c[:, :, None, :] * B_c[:, :, :, None] * Em[None]
            S = U
            shift = 1
            while shift < T:
                Sz = jnp.concatenate(
                    [jnp.zeros_like(S[:, :shift]), S[:, :-shift]], axis=1
                )
                S = S + Sz
                shift *= 2
            h = Ep[None] * S + Ep1[None] * h_prev[:, None]
            y_c = jnp.sum(h * C_c[:, :, :, None], axis=2)
            out_ref[:, sl, :] = y_c
            return h[:, T - 1]

        h_fin = lax.fori_loop(0, n_chunks, chunk, h0_ref[...])
        hf_ref[...] = h_fin

        @pl.when(my_x == 0)
        def _():
            send = pltpu.make_async_remote_copy(
                src_ref=hf_ref,
                dst_ref=h0_ref,
                send_sem=send_sem,
                recv_sem=recv_sem,
                device_id=nbr,
                device_id_type=pl.DeviceIdType.MESH,
            )
            send.start()
            send.wait_send()

    return pl.pallas_call(
        body,
        out_shape=jax.ShapeDtypeStruct((b, s_loc, d), jnp.float32),
        in_specs=[pl.BlockSpec(memory_space=pltpu.VMEM)] * 4,
        out_specs=pl.BlockSpec(memory_space=pltpu.VMEM),
        scratch_shapes=[
            pltpu.VMEM((b, n, d), jnp.float32),
            pltpu.VMEM((b, n, d), jnp.float32),
            pltpu.SemaphoreType.DMA,
            pltpu.SemaphoreType.DMA,
        ],
        compiler_params=pltpu.CompilerParams(
            collective_id=0, vmem_limit_bytes=110 * 1024 * 1024
        ),
    )(x, A, B, C)


# device time: 122550 ns/iter; 1.3785x vs baseline; 1.3785x over previous
import jax
import jax.numpy as jnp
from jax import lax
from jax.experimental import pallas as pl
from jax.experimental.pallas import tpu as pltpu

T = 32


def kernel(x, A, B, C):
    b, s_loc, d = x.shape
    n = B.shape[-1]
    n_chunks = s_loc // T

    def body(x_ref, A_ref, B_ref, C_ref, out_ref, h0_ref, hf_ref, send_sem, recv_sem):
        my_x = lax.axis_index("x")
        my_y = lax.axis_index("y")
        nbr = (1 - my_x, my_y)

        barrier = pltpu.get_barrier_semaphore()
        pl.semaphore_signal(
            barrier, inc=1, device_id=nbr, device_id_type=pl.DeviceIdType.MESH
        )
        pl.semaphore_wait(barrier, 1)

        At = A_ref[:, :].astype(jnp.float32).T
        tau = lax.broadcasted_iota(jnp.int32, (T, 1, 1), 0).astype(jnp.float32)
        Ep = jnp.exp(At[None] * tau)
        Em = jnp.exp(At[None] * (-tau))
        dA = jnp.exp(At)
        Ep1 = Ep * dA[None]
        EpT = jnp.exp(At * float(T))

        def chunk(k, h_prev):
            sl = pl.ds(k * T, T)
            x_c = x_ref[:, sl, :].astype(jnp.float32)
            B_c = B_ref[:, sl, :].astype(jnp.float32)
            C_c = C_ref[:, sl, :].astype(jnp.float32)
            U = x_c[:, :, None, :] * B_c[:, :, :, None] * Em[None]
            S = U
            shift = 1
            while shift < T:
                Sz = jnp.concatenate(
                    [jnp.zeros_like(S[:, :shift]), S[:, :-shift]], axis=1
                )
                S = S + Sz
                shift *= 2
            h = Ep[None] * (S + (dA * h_prev)[:, None])
            y_c = jnp.sum(h * C_c[:, :, :, None], axis=2)
            out_ref[:, sl, :] = y_c
            return h[:, T - 1]

        h_fin = lax.fori_loop(
            0, n_chunks, chunk, jnp.zeros((b, n, d), jnp.float32)
        )
        hf_ref[...] = h_fin

        @pl.when(my_x == 0)
        def _():
            send = pltpu.make_async_remote_copy(
                src_ref=hf_ref,
                dst_ref=h0_ref,
                send_sem=send_sem,
                recv_sem=recv_sem,
                device_id=nbr,
                device_id_type=pl.DeviceIdType.MESH,
            )
            send.start()
            send.wait_send()

        @pl.when(my_x == 1)
        def _():
            recv = pltpu.make_async_remote_copy(
                src_ref=hf_ref,
                dst_ref=h0_ref,
                send_sem=send_sem,
                recv_sem=recv_sem,
                device_id=nbr,
                device_id_type=pl.DeviceIdType.MESH,
            )
            recv.wait_recv()

            def corr(k, hc):
                sl = pl.ds(k * T, T)
                C_c = C_ref[:, sl, :].astype(jnp.float32)
                hterm = Ep1[None] * hc[:, None]
                y_add = jnp.sum(hterm * C_c[:, :, :, None], axis=2)
                out_ref[:, sl, :] += y_add
                return EpT * hc

            lax.fori_loop(0, n_chunks, corr, h0_ref[...])

    return pl.pallas_call(
        body,
        out_shape=jax.ShapeDtypeStruct((b, s_loc, d), jnp.float32),
        in_specs=[pl.BlockSpec(memory_space=pltpu.VMEM)] * 4,
        out_specs=pl.BlockSpec(memory_space=pltpu.VMEM),
        scratch_shapes=[
            pltpu.VMEM((b, n, d), jnp.float32),
            pltpu.VMEM((b, n, d), jnp.float32),
            pltpu.SemaphoreType.DMA,
            pltpu.SemaphoreType.DMA,
        ],
        compiler_params=pltpu.CompilerParams(
            collective_id=0, vmem_limit_bytes=110 * 1024 * 1024
        ),
    )(x, A, B, C)
